# Initial kernel scaffold; baseline (speedup 1.0000x reference)
import jax
import jax.numpy as jnp
from jax.experimental import pallas as pl


def kernel(x, edge_index, edge_weight, W_in, b_in, Wg0, bg0, ln0_g, ln0_b, Wg1, bg1, ln1_g, ln1_b, Wih0, Whh0, bih0, bhh0, Wih1, Whh1, bih1, bhh1, fc1_W, fc1_b, fc2_W, fc2_b):
    raise NotImplementedError("write your pallas kernel here")


# speedup vs baseline: 15.4780x; 15.4780x over previous
import jax
import jax.numpy as jnp
from jax import lax
from jax.experimental import pallas as pl
from jax.experimental.pallas import tpu as pltpu
from jax.experimental.pallas import tpu_sc as plsc

NC = 2
NS = 16
NW = NC * NS
L = 16
K = 80
SB = 25
G = 2
ZR = 16


def _dot_t(a, b):
    return lax.dot_general(a, b, (((1,), (1,)), ((), ())),
                           preferred_element_type=jnp.float32)


def _make_deg_kernel(NP, nchunk):
    rpt = NP // NS
    mesh = plsc.VectorSubcoreMesh(core_axis_name="c", subcore_axis_name="s")

    def body(dst_hbm, w_hbm, out_hbm, dst_v, w_v, bufd, zd, accd):
        cid = lax.axis_index("c")
        sid = lax.axis_index("s")
        wid = cid * NS + sid
        pltpu.sync_copy(dst_hbm.at[wid], dst_v)
        pltpu.sync_copy(w_hbm.at[wid], w_v)

        @pl.loop(0, ZR)
        def _zero(i):
            zd[i, :] = jnp.zeros((L,), jnp.float32)

        for q in range(rpt // ZR):
            pltpu.sync_copy(zd, accd.at[pl.ds(sid * rpt + q * ZR, ZR)])
        plsc.subcore_barrier()

        @pl.loop(0, nchunk)
        def _chunk(c):
            @pl.loop(0, K // L)
            def _grp(u):
                base = u * L
                w16 = w_v[c, pl.ds(base, L)]
                for j in range(L):
                    bufd[base + j, :] = w16 * 0.0 + w16[j]
            pltpu.sync_copy(bufd, accd.at[dst_v.at[c]], add=True)

        plsc.subcore_barrier()
        pltpu.sync_copy(accd.at[pl.ds(sid * rpt, rpt)],
                        out_hbm.at[cid].at[pl.ds(sid * rpt, rpt)])

    return pl.kernel(
        body,
        out_type=jax.ShapeDtypeStruct((NC, NP, L), jnp.float32),
        mesh=mesh,
        scratch_types=[
            pltpu.VMEM((nchunk, K), jnp.int32),
            pltpu.VMEM((nchunk, K), jnp.float32),
            pltpu.VMEM((K, L), jnp.float32),
            pltpu.VMEM((ZR, L), jnp.float32),
            pltpu.VMEM_SHARED((NP, L), jnp.float32),
        ],
    )


def _make_spmm_kernel(NP, DG, NG, nchunk):
    rpt = NP // NS
    nsb = nchunk // SB
    mesh = plsc.VectorSubcoreMesh(core_axis_name="c", subcore_axis_name="s")

    def body(table_hbm, e_hbm, out_hbm,
             ib0, ib1, buf0, buf1, zrow, acc, semi0, semi1, sem0, sem1):
        cid = lax.axis_index("c")
        sid = lax.axis_index("s")
        wid = cid * NS + sid

        @pl.loop(0, ZR)
        def _zero(i):
            for v in range(DG // L):
                zrow[i, pl.ds(v * L, L)] = jnp.zeros((L,), jnp.float32)

        def scale_scatter(ib, c, buf):
            @pl.loop(0, K // L)
            def _grp(u):
                base = u * L
                w16 = lax.bitcast_convert_type(
                    ib[c, 2, pl.ds(base, L)], jnp.float32)
                for j in range(L):
                    nr = w16[j]
                    r = base + j
                    for v in range(DG // L):
                        sl = pl.ds(v * L, L)
                        buf[r, sl] = buf[r, sl] * nr
            pltpu.sync_copy(buf, acc.at[ib.at[c, 1]], add=True)

        @pl.loop(0, NG)
        def _group(g):
            for q in range(rpt // ZR):
                pltpu.sync_copy(zrow, acc.at[pl.ds(sid * rpt + q * ZR, ZR)])
            plsc.subcore_barrier()

            tbl = table_hbm.at[g]
            pltpu.async_copy(e_hbm.at[wid, pl.ds(0, SB)], ib0, semi0)
            for sb in range(nsb):
                ib, semi = (ib0, semi0) if sb % 2 == 0 else (ib1, semi1)
                nib, nsemi = (ib1, semi1) if sb % 2 == 0 else (ib0, semi0)
                pltpu.make_async_copy(
                    e_hbm.at[wid, pl.ds(sb * SB, SB)], ib, semi).wait()
                if sb + 1 < nsb:
                    pltpu.async_copy(
                        e_hbm.at[wid, pl.ds((sb + 1) * SB, SB)], nib, nsemi)

                pltpu.async_copy(tbl.at[ib.at[0, 0]], buf0, sem0)

                @pl.loop(0, (SB - 1) // 2)
                def _pair(i):
                    a = 2 * i
                    b = a + 1
                    pltpu.async_copy(tbl.at[ib.at[b, 0]], buf1, sem1)
                    pltpu.make_async_copy(tbl.at[ib.at[a, 0]], buf0,
                                          sem0).wait()
                    scale_scatter(ib, a, buf0)
                    pltpu.async_copy(tbl.at[ib.at[b + 1, 0]], buf0, sem0)
                    pltpu.make_async_copy(tbl.at[ib.at[b, 0]], buf1,
                                          sem1).wait()
                    scale_scatter(ib, b, buf1)

                last = SB - 1
                pltpu.make_async_copy(tbl.at[ib.at[last, 0]], buf0,
                                      sem0).wait()
                scale_scatter(ib, last, buf0)

            plsc.subcore_barrier()
            pltpu.sync_copy(acc.at[pl.ds(sid * rpt, rpt)],
                            out_hbm.at[cid * NG + g].at[pl.ds(sid * rpt, rpt)])
            plsc.subcore_barrier()

    return pl.kernel(
        body,
        out_type=jax.ShapeDtypeStruct((NC * NG, NP, DG), jnp.float32),
        mesh=mesh,
        scratch_types=[
            pltpu.VMEM((SB, 3, K), jnp.int32),
            pltpu.VMEM((SB, 3, K), jnp.int32),
            pltpu.VMEM((K, DG), jnp.float32),
            pltpu.VMEM((K, DG), jnp.float32),
            pltpu.VMEM((ZR, DG), jnp.float32),
            pltpu.VMEM_SHARED((NP, DG), jnp.float32),
            pltpu.SemaphoreType.DMA,
            pltpu.SemaphoreType.DMA,
            pltpu.SemaphoreType.DMA,
            pltpu.SemaphoreType.DMA,
        ],
    )


def _dinv_body(dp_ref, out_ref):
    r = lax.rsqrt(dp_ref[0] + dp_ref[1] + 1.0)
    out_ref[...] = jnp.concatenate([r, r, r, r], axis=-1)


def _a_body(x_ref, win_ref, bin_ref, wg0_ref, dinv_ref, out_ref):
    di = dinv_ref[...]
    for s in range(G):
        h = jnp.maximum(_dot_t(x_ref[s], win_ref[...]) + bin_ref[...], 0.0)
        out_ref[0, :, s, :] = _dot_t(h, wg0_ref[...]) * di


def _layernorm_relu(z, lng, lnb):
    mu = jnp.mean(z, axis=-1, keepdims=True)
    zc = z - mu
    var = jnp.mean(zc * zc, axis=-1, keepdims=True)
    zn = zc * lax.rsqrt(var + 1e-5) * lng + lnb
    return jnp.maximum(zn, 0.0)


def _make_b_body(NG):
    def body(s_ref, g0_ref, dinv_ref, b_ref, lng_ref, lnb_ref, wg1_ref,
             out_ref):
        di = dinv_ref[...]
        for g in range(NG):
            for s in range(G):
                agg = (s_ref[g, :, s, :] + s_ref[NG + g, :, s, :] +
                       g0_ref[g, :, s, :])
                z = di * agg + b_ref[...]
                h = _layernorm_relu(z, lng_ref[...], lnb_ref[...])
                out_ref[g, :, s, :] = _dot_t(h, wg1_ref[...]) * di
    return body


def _lstm_step(xt, h, c, wih, whh, bih, bhh):
    gates = _dot_t(xt, wih) + bih + _dot_t(h, whh) + bhh
    H = h.shape[-1]
    i = jax.nn.sigmoid(gates[:, 0:H])
    f = jax.nn.sigmoid(gates[:, H:2 * H])
    gg = jnp.tanh(gates[:, 2 * H:3 * H])
    o = jax.nn.sigmoid(gates[:, 3 * H:4 * H])
    c = f * c + i * gg
    h = o * jnp.tanh(c)
    return h, c


def _make_cd_body(T, NG, H):
    def body(s_ref, g1_ref, dinv_ref, b_ref, lng_ref, lnb_ref,
             wih0_ref, whh0_ref, bih0_ref, bhh0_ref,
             wih1_ref, whh1_ref, bih1_ref, bhh1_ref,
             fc1w_ref, fc1b_ref, fc2w_ref, fc2b_ref, out_ref):
        di = dinv_ref[...]
        bn = di.shape[0]
        h0 = jnp.zeros((bn, H), jnp.float32)
        c0 = jnp.zeros((bn, H), jnp.float32)
        h1 = jnp.zeros((bn, H), jnp.float32)
        c1 = jnp.zeros((bn, H), jnp.float32)
        for t in range(T):
            g, s = t // G, t % G
            agg = (s_ref[g, :, s, :] + s_ref[NG + g, :, s, :] +
                   g1_ref[g, :, s, :])
            z = di * agg + b_ref[...]
            xt = _layernorm_relu(z, lng_ref[...], lnb_ref[...])
            h0, c0 = _lstm_step(xt, h0, c0, wih0_ref[...], whh0_ref[...],
                                bih0_ref[...], bhh0_ref[...])
            h1, c1 = _lstm_step(h0, h1, c1, wih1_ref[...], whh1_ref[...],
                                bih1_ref[...], bhh1_ref[...])
        y = jnp.maximum(_dot_t(h1, fc1w_ref[...]) + fc1b_ref[...], 0.0)
        out_ref[...] = (jnp.sum(y * fc2w_ref[...], axis=-1, keepdims=True) +
                        fc2b_ref[...])
    return body


def kernel(x, edge_index, edge_weight, W_in, b_in, Wg0, bg0, ln0_g, ln0_b,
           Wg1, bg1, ln1_g, ln1_b, Wih0, Whh0, bih0, bhh0, Wih1, Whh1,
           bih1, bhh1, fc1_W, fc1_b, fc2_W, fc2_b):
    B, T, N, F = x.shape
    H = W_in.shape[0]
    E = edge_index.shape[1]
    NG = T // G
    DG = G * H
    BN = 400
    NB = N // BN
    NP = -(-N // (8 * NS)) * (8 * NS)

    blk = NW * K * SB
    epad = (-E) % blk
    src = edge_index[0]
    dst = edge_index[1]
    ew = edge_weight
    if epad:
        src = jnp.concatenate([src, jnp.zeros((epad,), jnp.int32)])
        dst = jnp.concatenate([dst, jnp.zeros((epad,), jnp.int32)])
        ew = jnp.concatenate([ew, jnp.zeros((epad,), jnp.float32)])
    nchunk = (E + epad) // (NW * K)
    srcr = src.reshape(NW, nchunk, K)
    dstr = dst.reshape(NW, nchunk, K)
    ewr = ew.reshape(NW, nchunk, K)
    edges = jnp.stack(
        [srcr, dstr, lax.bitcast_convert_type(ewr, jnp.int32)], axis=2)

    x3 = x.reshape(T, N, F)
    b_in2 = b_in.reshape(1, H)
    bg0_2 = bg0.reshape(1, H)
    bg1_2 = bg1.reshape(1, H)
    ln0g2, ln0b2 = ln0_g.reshape(1, H), ln0_b.reshape(1, H)
    ln1g2, ln1b2 = ln1_g.reshape(1, H), ln1_b.reshape(1, H)
    bih02, bhh02 = bih0.reshape(1, 4 * H), bhh0.reshape(1, 4 * H)
    bih12, bhh12 = bih1.reshape(1, 4 * H), bhh1.reshape(1, 4 * H)
    fc1b2 = fc1_b.reshape(1, H // 2)
    fc2b2 = fc2_b.reshape(1, 1)

    degp = _make_deg_kernel(NP, nchunk)(dstr, ewr)
    dinv2 = pl.pallas_call(
        _dinv_body,
        grid=(NB,),
        in_specs=[pl.BlockSpec((NC, BN, L), lambda i: (0, i, 0))],
        out_specs=pl.BlockSpec((BN, H), lambda i: (i, 0)),
        out_shape=jax.ShapeDtypeStruct((N, H), jnp.float32),
    )(degp)

    g0 = pl.pallas_call(
        _a_body,
        grid=(NG, NB),
        in_specs=[
            pl.BlockSpec((G, BN, F), lambda g, i: (g, i, 0)),
            pl.BlockSpec((H, F), lambda g, i: (0, 0)),
            pl.BlockSpec((1, H), lambda g, i: (0, 0)),
            pl.BlockSpec((H, H), lambda g, i: (0, 0)),
            pl.BlockSpec((BN, H), lambda g, i: (i, 0)),
        ],
        out_specs=pl.BlockSpec((1, BN, G, H), lambda g, i: (g, i, 0, 0)),
        out_shape=jax.ShapeDtypeStruct((NG, NP, G, H), jnp.float32),
    )(x3, W_in, b_in2, Wg0, dinv2)

    spmm = _make_spmm_kernel(NP, DG, NG, nchunk)

    s0 = spmm(g0.reshape(NG, NP, DG), edges)

    g1 = pl.pallas_call(
        _make_b_body(NG),
        grid=(NB,),
        in_specs=[
            pl.BlockSpec((NC * NG, BN, G, H), lambda i: (0, i, 0, 0)),
            pl.BlockSpec((NG, BN, G, H), lambda i: (0, i, 0, 0)),
            pl.BlockSpec((BN, H), lambda i: (i, 0)),
            pl.BlockSpec((1, H), lambda i: (0, 0)),
            pl.BlockSpec((1, H), lambda i: (0, 0)),
            pl.BlockSpec((1, H), lambda i: (0, 0)),
            pl.BlockSpec((H, H), lambda i: (0, 0)),
        ],
        out_specs=pl.BlockSpec((NG, BN, G, H), lambda i: (0, i, 0, 0)),
        out_shape=jax.ShapeDtypeStruct((NG, NP, G, H), jnp.float32),
    )(s0.reshape(NC * NG, NP, G, H), g0, dinv2, bg0_2, ln0g2, ln0b2, Wg1)

    s1 = spmm(g1.reshape(NG, NP, DG), edges)

    full = lambda shape: pl.BlockSpec(shape, lambda i: tuple(0 for _ in shape))
    out = pl.pallas_call(
        _make_cd_body(T, NG, H),
        grid=(NB,),
        in_specs=[
            pl.BlockSpec((NC * NG, BN, G, H), lambda i: (0, i, 0, 0)),
            pl.BlockSpec((NG, BN, G, H), lambda i: (0, i, 0, 0)),
            pl.BlockSpec((BN, H), lambda i: (i, 0)),
            full((1, H)), full((1, H)), full((1, H)),
            full((4 * H, H)), full((4 * H, H)), full((1, 4 * H)), full((1, 4 * H)),
            full((4 * H, H)), full((4 * H, H)), full((1, 4 * H)), full((1, 4 * H)),
            full((H // 2, H)), full((1, H // 2)), full((1, H // 2)), full((1, 1)),
        ],
        out_specs=pl.BlockSpec((BN, 1), lambda i: (i, 0)),
        out_shape=jax.ShapeDtypeStruct((N, 1), jnp.float32),
    )(s1.reshape(NC * NG, NP, G, H), g1, dinv2, bg1_2, ln1g2, ln1b2,
      Wih0, Whh0, bih02, bhh02, Wih1, Whh1, bih12, bhh12,
      fc1_W, fc1b2, fc2_W, fc2b2)

    return out.reshape(B, N, 1)
